# Initial kernel scaffold; baseline (speedup 1.0000x reference)
import jax
import jax.numpy as jnp
from jax.experimental import pallas as pl


def kernel(z, pos_embed):
    raise NotImplementedError("write your pallas kernel here")


# speedup vs baseline: 1.3873x; 1.3873x over previous
import jax
import jax.numpy as jnp
from jax.experimental import pallas as pl

SEQ_LEN = 2048
D_MODEL = 1024
BATCH = 32
BLOCK_S = 64


def _bcast_body(pe_ref, out_ref):
    pe = pe_ref[...]
    out_ref[...] = jnp.broadcast_to(pe[:, None, :], (BLOCK_S, BATCH, D_MODEL))


def kernel(z, pos_embed):
    del z
    grid = (SEQ_LEN // BLOCK_S,)
    out = pl.pallas_call(
        _bcast_body,
        grid=grid,
        in_specs=[pl.BlockSpec((BLOCK_S, D_MODEL), lambda i: (i, 0))],
        out_specs=pl.BlockSpec((BLOCK_S, BATCH, D_MODEL), lambda i: (i, 0, 0)),
        out_shape=jax.ShapeDtypeStruct((SEQ_LEN, BATCH, D_MODEL), jnp.float32),
    )(pos_embed)
    return out
